# baseline (speedup 1.0000x reference)
import functools

import jax
import jax.numpy as jnp
from jax import lax
from jax.experimental import pallas as pl
from jax.experimental.pallas import tpu as pltpu
from jax.experimental.pallas import tpu_sc as plsc

L = 16
NW = 32
D = 256
DV = D // L
C = 64
PB = 32


def _decoder_body(E, z_hbm, idx_hbm, out_hbm,
                  cidx_v, g0_v, pbuf_v, outv_v, ss0, ss1):
    nchunk = E // C
    bnk = nchunk // NW
    rem = nchunk - bnk * NW
    maxnk = bnk + (1 if rem else 0)
    wid = lax.axis_index("c") * 16 + lax.axis_index("s")
    nk = jnp.where(wid < rem, bnk + 1, bnk)
    start_chunk = wid * bnk + jnp.minimum(wid, rem)
    ebase = start_chunk * C

    pltpu.sync_copy(idx_hbm.at[pl.ds(ebase * 2, maxnk * 2 * C)], cidx_v)

    def start(c):
        par = c % 2

        @pl.when(par == 0)
        def _():
            pltpu.async_copy(z_hbm.at[cidx_v.at[pl.ds(c * 2 * C, 2 * C)]],
                             g0_v.at[pl.ds(0, 2 * C)], ss0)

        @pl.when(par == 1)
        def _():
            pltpu.async_copy(z_hbm.at[cidx_v.at[pl.ds(c * 2 * C, 2 * C)]],
                             g0_v.at[pl.ds(2 * C, 2 * C)], ss1)

    def wait(c):
        par = c % 2

        @pl.when(par == 0)
        def _():
            pltpu.make_async_copy(z_hbm.at[cidx_v.at[pl.ds(c * 2 * C, 2 * C)]],
                                  g0_v.at[pl.ds(0, 2 * C)], ss0).wait()

        @pl.when(par == 1)
        def _():
            pltpu.make_async_copy(z_hbm.at[cidx_v.at[pl.ds(c * 2 * C, 2 * C)]],
                                  g0_v.at[pl.ds(2 * C, 2 * C)], ss1).wait()

    lanes = lax.broadcasted_iota(jnp.int32, (L,), 0)
    maskf = [jnp.where(lanes == e, 1.0, 0.0).astype(jnp.float32)
             for e in range(L)]

    def compute(c):
        rowb = (c % 2) * 2 * C

        def group_body(g, _):
            def edge_body(e, res):
                srow = rowb + g * L + e
                drow = srow + C
                acc = (g0_v[srow, pl.ds(0, L)] * g0_v[drow, pl.ds(0, L)])
                for i in range(1, DV):
                    acc = acc + (g0_v[srow, pl.ds(i * L, L)]
                                 * g0_v[drow, pl.ds(i * L, L)])
                pbuf_v[pl.ds(e * PB, L)] = acc
                for h in (8, 4, 2):
                    acc = acc + pbuf_v[pl.ds(e * PB + h, L)]
                    pbuf_v[pl.ds(e * PB, L)] = acc
                return jnp.where(lanes == e,
                                 pbuf_v[pl.ds((PB - 1) * e, L)]
                                 + pbuf_v[pl.ds((PB - 1) * e + 1, L)], res)

            res = lax.fori_loop(0, L, edge_body,
                                jnp.zeros((L,), jnp.float32))
            outv_v[pl.ds(g * L, L)] = 1.0 / (1.0 + jnp.exp(-res))
            return 0

        lax.fori_loop(0, C // L, group_body, 0)
        pltpu.sync_copy(outv_v, out_hbm.at[pl.ds(ebase + c * C, C)])

    start(0)

    def pipe_body(c, _):
        @pl.when(c + 1 < nk)
        def _():
            start(c + 1)

        @pl.when(c < nk)
        def _():
            wait(c)
            compute(c)

        return 0

    lax.fori_loop(0, maxnk, pipe_body, 0)


def kernel(z, edge_index):
    E = edge_index.shape[1]
    nchunk = E // C
    bnk = nchunk // NW
    rem = nchunk % NW
    maxnk = bnk + (1 if rem else 0)
    npadc = nchunk + (1 if rem else 0)
    pad = npadc * C - E
    src = jnp.pad(edge_index[0], (0, pad)).reshape(npadc, C)
    dst = jnp.pad(edge_index[1], (0, pad)).reshape(npadc, C)
    idx = jnp.stack([src, dst], axis=1).reshape(-1)

    mesh = plsc.VectorSubcoreMesh(core_axis_name="c", subcore_axis_name="s")
    body = functools.partial(_decoder_body, E)
    f = pl.kernel(
        body,
        out_type=jax.ShapeDtypeStruct((E,), jnp.float32),
        mesh=mesh,
        scratch_types=[
            pltpu.VMEM((maxnk * 2 * C,), jnp.int32),
            pltpu.VMEM((4 * C, D), jnp.float32),
            pltpu.VMEM((L * PB + L,), jnp.float32),
            pltpu.VMEM((C,), jnp.float32),
            pltpu.SemaphoreType.DMA,
            pltpu.SemaphoreType.DMA,
        ],
    )
    return f(z, idx)
